# baseline (speedup 1.0000x reference)
import functools

import jax
import jax.numpy as jnp
from jax import lax
from jax.experimental import pallas as pl
from jax.experimental.pallas import tpu as pltpu
from jax.experimental.pallas import tpu_sc as plsc

D = 128
S = 64
T = 50
F = 4


def _sc_gather(table, idx_flat):
    info = plsc.get_sparse_core_info()
    nc, ns = info.num_cores, info.num_subcores
    nw = nc * ns
    n = idx_flat.shape[0]
    d = table.shape[1]
    b_per_w = n // nw
    chunk = 400
    n_chunks = b_per_w // chunk
    mesh = plsc.VectorSubcoreMesh(core_axis_name="c", subcore_axis_name="s")

    @functools.partial(
        pl.kernel,
        mesh=mesh,
        out_type=jax.ShapeDtypeStruct((n, d), jnp.float32),
        scratch_types=[
            pltpu.VMEM((b_per_w,), jnp.int32),
            pltpu.VMEM((chunk, d), jnp.float32),
            pltpu.SemaphoreType.DMA,
        ],
    )
    def k(table_hbm, idx_hbm, out_hbm, idx_v, rows_v, sem):
        wid = lax.axis_index("s") * nc + lax.axis_index("c")
        base = wid * b_per_w
        pltpu.sync_copy(idx_hbm.at[pl.ds(base, b_per_w)], idx_v)
        for c in range(n_chunks):
            pltpu.async_copy(
                table_hbm.at[idx_v.at[pl.ds(c * chunk, chunk)]], rows_v, sem
            ).wait()
            pltpu.sync_copy(rows_v, out_hbm.at[pl.ds(base + c * chunk, chunk)])

    return k(table, idx_flat)


def _enc_body(x_ref, w1_ref, b1_ref, w2_ref, b2_ref, g_ref, be_ref, wg_ref,
              bg_ref, hid_ref, si_ref):
    x = x_ref[...]
    h1 = jnp.dot(x, w1_ref[...], preferred_element_type=jnp.float32) + b1_ref[...]
    h1 = jnp.maximum(h1, 0.0)
    ff = jnp.dot(h1, w2_ref[...], preferred_element_type=jnp.float32) + b2_ref[...]
    y = x + ff
    mu = jnp.mean(y, axis=1, keepdims=True)
    yc = y - mu
    var = jnp.mean(yc * yc, axis=1, keepdims=True)
    hid = yc / jnp.sqrt(var + 1e-5) * g_ref[...] + be_ref[...]
    hid_ref[...] = hid
    gate = jnp.dot(hid, wg_ref[...], preferred_element_type=jnp.float32) + bg_ref[...]
    si_ref[...] = jnp.mean(gate, axis=1, keepdims=True)


def _encoder(h_flat, w1, b1, w2, b2, gamma, beta, wg, bg):
    n = h_flat.shape[0]
    rblk = 6400
    grid = n // rblk
    return pl.pallas_call(
        _enc_body,
        grid=(grid,),
        in_specs=[
            pl.BlockSpec((rblk, D), lambda i: (i, 0)),
            pl.BlockSpec((D, 2 * D), lambda i: (0, 0)),
            pl.BlockSpec((1, 2 * D), lambda i: (0, 0)),
            pl.BlockSpec((2 * D, D), lambda i: (0, 0)),
            pl.BlockSpec((1, D), lambda i: (0, 0)),
            pl.BlockSpec((1, D), lambda i: (0, 0)),
            pl.BlockSpec((1, D), lambda i: (0, 0)),
            pl.BlockSpec((D, S), lambda i: (0, 0)),
            pl.BlockSpec((1, S), lambda i: (0, 0)),
        ],
        out_specs=[
            pl.BlockSpec((rblk, D), lambda i: (i, 0)),
            pl.BlockSpec((rblk, 1), lambda i: (i, 0)),
        ],
        out_shape=[
            jax.ShapeDtypeStruct((n, D), jnp.float32),
            jax.ShapeDtypeStruct((n, 1), jnp.float32),
        ],
    )(h_flat, w1, b1.reshape(1, -1), w2, b2.reshape(1, -1),
      gamma.reshape(1, -1), beta.reshape(1, -1), wg, bg.reshape(1, -1))


def _sel_body(hid_ref, si_ref, wr_ref, br_ref, mem_ref, ctx_ref):
    hid = hid_ref[...]
    hk = hid[:, : T - 1, :]
    bb = hid.shape[0]
    tk = T - 1

    si = si_ref[...][:, : T - 1]

    q = jnp.dot(hid[:, T - 1, :], wr_ref[...],
                preferred_element_type=jnp.float32) + br_ref[...]

    iota_t = lax.broadcasted_iota(jnp.int32, (bb, tk), 1)
    si_w = si
    rows = []
    score_cols = []
    for _ in range(F):
        m = jnp.max(si_w, axis=1, keepdims=True)
        first = jnp.min(jnp.where(si_w == m, iota_t, tk), axis=1, keepdims=True)
        onehot = iota_t == first
        row = jnp.sum(hk * onehot.astype(jnp.float32)[:, :, None], axis=1)
        rows.append(row)
        score_cols.append(jnp.sum(row * q, axis=1, keepdims=True))
        si_w = jnp.where(onehot, -1e30, si_w)

    mem_rest = mem_ref[...][F:, :]
    scores_rest = lax.dot_general(
        q, mem_rest, (((1,), (1,)), ((), ())),
        preferred_element_type=jnp.float32)
    scores = jnp.concatenate(score_cols + [scores_rest], axis=1)
    mx = jnp.max(scores, axis=1, keepdims=True)
    e = jnp.exp(scores - mx)
    attn = e / jnp.sum(e, axis=1, keepdims=True)

    ctx = jnp.dot(attn[:, F:], mem_rest, preferred_element_type=jnp.float32)
    for i in range(F):
        ctx = ctx + attn[:, i : i + 1] * rows[i]
    ctx_ref[...] = ctx


def _select_read(hid3, si2, wr, br, mem_r):
    b = hid3.shape[0]
    bblk = 256
    return pl.pallas_call(
        _sel_body,
        grid=(b // bblk,),
        in_specs=[
            pl.BlockSpec((bblk, T, D), lambda i: (i, 0, 0)),
            pl.BlockSpec((bblk, T), lambda i: (i, 0)),
            pl.BlockSpec((D, D), lambda i: (0, 0)),
            pl.BlockSpec((1, D), lambda i: (0, 0)),
            pl.BlockSpec((S, D), lambda i: (0, 0)),
        ],
        out_specs=pl.BlockSpec((bblk, D), lambda i: (i, 0)),
        out_shape=jax.ShapeDtypeStruct((b, D), jnp.float32),
    )(hid3, si2, wr, br.reshape(1, -1), mem_r)


def _proj_body(ctx_ref, wo_ref, bo_ref, out_ref):
    out_ref[...] = (
        jnp.dot(ctx_ref[...].astype(jnp.bfloat16), wo_ref[...],
                preferred_element_type=jnp.float32)
        + bo_ref[...]
    )


def _project(ctx, wo, bo):
    b = ctx.shape[0]
    v = wo.shape[1]
    vblk = 2048
    grid = pl.cdiv(v, vblk)
    return pl.pallas_call(
        _proj_body,
        grid=(grid,),
        in_specs=[
            pl.BlockSpec((b, D), lambda j: (0, 0)),
            pl.BlockSpec((D, vblk), lambda j: (0, j)),
            pl.BlockSpec((1, vblk), lambda j: (0, j)),
        ],
        out_specs=pl.BlockSpec((b, vblk), lambda j: (0, j)),
        out_shape=jax.ShapeDtypeStruct((b, v), jnp.float32),
    )(ctx, wo.astype(jnp.bfloat16), bo.reshape(1, -1))


def kernel(seq, embed, W1, b1, W2, b2, gamma, beta, Wg, bg, Wr, br, Wo, bo, memory):
    b, t = seq.shape
    idx = seq.astype(jnp.int32).reshape(-1)
    h_flat = _sc_gather(embed, idx)
    hid, si = _encoder(h_flat, W1, b1, W2, b2, gamma, beta, Wg, bg)
    hid3 = hid.reshape(b, t, D)
    ctx = _select_read(hid3, si.reshape(b, t), Wr, br, memory[0])
    return _project(ctx, Wo, bo)


# speedup vs baseline: 1.7356x; 1.7356x over previous
import functools

import jax
import jax.numpy as jnp
from jax import lax
from jax.experimental import pallas as pl
from jax.experimental.pallas import tpu as pltpu
from jax.experimental.pallas import tpu_sc as plsc

D = 128
S = 64
T = 50
F = 4


def _sc_gather(table, idx_flat):
    info = plsc.get_sparse_core_info()
    nc, ns = info.num_cores, info.num_subcores
    nw = nc * ns
    n = idx_flat.shape[0]
    d = table.shape[1]
    b_per_w = n // nw
    chunk = 400
    n_chunks = b_per_w // chunk
    mesh = plsc.VectorSubcoreMesh(core_axis_name="c", subcore_axis_name="s")

    @functools.partial(
        pl.kernel,
        mesh=mesh,
        out_type=jax.ShapeDtypeStruct((n, d), jnp.float32),
        scratch_types=[
            pltpu.VMEM((b_per_w,), jnp.int32),
            pltpu.VMEM((chunk, d), jnp.float32),
            pltpu.SemaphoreType.DMA,
        ],
    )
    def k(table_hbm, idx_hbm, out_hbm, idx_v, rows_v, sem):
        wid = lax.axis_index("s") * nc + lax.axis_index("c")
        base = wid * b_per_w
        pltpu.sync_copy(idx_hbm.at[pl.ds(base, b_per_w)], idx_v)
        for c in range(n_chunks):
            pltpu.async_copy(
                table_hbm.at[idx_v.at[pl.ds(c * chunk, chunk)]], rows_v, sem
            ).wait()
            pltpu.sync_copy(rows_v, out_hbm.at[pl.ds(base + c * chunk, chunk)])

    return k(table, idx_flat)


def _enc_body(x_ref, w1_ref, b1_ref, w2_ref, b2_ref, g_ref, be_ref, wg_ref,
              bg_ref, hid_ref, si_ref):
    x = x_ref[...]
    h1 = jnp.dot(x, w1_ref[...], preferred_element_type=jnp.float32) + b1_ref[...]
    h1 = jnp.maximum(h1, 0.0)
    ff = jnp.dot(h1, w2_ref[...], preferred_element_type=jnp.float32) + b2_ref[...]
    y = x + ff
    mu = jnp.mean(y, axis=1, keepdims=True)
    yc = y - mu
    var = jnp.mean(yc * yc, axis=1, keepdims=True)
    hid = yc / jnp.sqrt(var + 1e-5) * g_ref[...] + be_ref[...]
    hid_ref[...] = hid
    gate = jnp.dot(hid, wg_ref[...], preferred_element_type=jnp.float32) + bg_ref[...]
    si_ref[...] = jnp.mean(gate, axis=1, keepdims=True)


def _encoder(h_flat, w1, b1, w2, b2, gamma, beta, wg, bg):
    n = h_flat.shape[0]
    rblk = 6400
    grid = n // rblk
    return pl.pallas_call(
        _enc_body,
        grid=(grid,),
        in_specs=[
            pl.BlockSpec((rblk, D), lambda i: (i, 0)),
            pl.BlockSpec((D, 2 * D), lambda i: (0, 0)),
            pl.BlockSpec((1, 2 * D), lambda i: (0, 0)),
            pl.BlockSpec((2 * D, D), lambda i: (0, 0)),
            pl.BlockSpec((1, D), lambda i: (0, 0)),
            pl.BlockSpec((1, D), lambda i: (0, 0)),
            pl.BlockSpec((1, D), lambda i: (0, 0)),
            pl.BlockSpec((D, S), lambda i: (0, 0)),
            pl.BlockSpec((1, S), lambda i: (0, 0)),
        ],
        out_specs=[
            pl.BlockSpec((rblk, D), lambda i: (i, 0)),
            pl.BlockSpec((rblk, 1), lambda i: (i, 0)),
        ],
        out_shape=[
            jax.ShapeDtypeStruct((n, D), jnp.float32),
            jax.ShapeDtypeStruct((n, 1), jnp.float32),
        ],
    )(h_flat, w1, b1.reshape(1, -1), w2, b2.reshape(1, -1),
      gamma.reshape(1, -1), beta.reshape(1, -1), wg, bg.reshape(1, -1))


def _sel_body(hid_ref, si_ref, wr_ref, br_ref, mem_ref, ctx_ref):
    hid = hid_ref[...]
    hk = hid[:, : T - 1, :]
    bb = hid.shape[0]
    tk = T - 1

    si = si_ref[...][:, : T - 1]

    q = jnp.dot(hid[:, T - 1, :], wr_ref[...],
                preferred_element_type=jnp.float32) + br_ref[...]

    iota_t = lax.broadcasted_iota(jnp.int32, (bb, tk), 1)
    si_w = si
    rows = []
    score_cols = []
    for _ in range(F):
        m = jnp.max(si_w, axis=1, keepdims=True)
        first = jnp.min(jnp.where(si_w == m, iota_t, tk), axis=1, keepdims=True)
        onehot = iota_t == first
        row = jnp.sum(hk * onehot.astype(jnp.float32)[:, :, None], axis=1)
        rows.append(row)
        score_cols.append(jnp.sum(row * q, axis=1, keepdims=True))
        si_w = jnp.where(onehot, -1e30, si_w)

    mem_rest = mem_ref[...][F:, :]
    scores_rest = lax.dot_general(
        q, mem_rest, (((1,), (1,)), ((), ())),
        preferred_element_type=jnp.float32)
    scores = jnp.concatenate(score_cols + [scores_rest], axis=1)
    mx = jnp.max(scores, axis=1, keepdims=True)
    e = jnp.exp(scores - mx)
    attn = e / jnp.sum(e, axis=1, keepdims=True)

    ctx = jnp.dot(attn[:, F:], mem_rest, preferred_element_type=jnp.float32)
    for i in range(F):
        ctx = ctx + attn[:, i : i + 1] * rows[i]
    ctx_ref[...] = ctx


def _select_read(hid3, si2, wr, br, mem_r):
    b = hid3.shape[0]
    bblk = 256
    return pl.pallas_call(
        _sel_body,
        grid=(b // bblk,),
        in_specs=[
            pl.BlockSpec((bblk, T, D), lambda i: (i, 0, 0)),
            pl.BlockSpec((bblk, T), lambda i: (i, 0)),
            pl.BlockSpec((D, D), lambda i: (0, 0)),
            pl.BlockSpec((1, D), lambda i: (0, 0)),
            pl.BlockSpec((S, D), lambda i: (0, 0)),
        ],
        out_specs=pl.BlockSpec((bblk, D), lambda i: (i, 0)),
        out_shape=jax.ShapeDtypeStruct((b, D), jnp.float32),
    )(hid3, si2, wr, br.reshape(1, -1), mem_r)


def _proj_body(ctx_ref, wo_ref, bo_ref, out_ref):
    res = lax.dot_general(
        wo_ref[...], ctx_ref[...].astype(jnp.bfloat16),
        (((0,), (1,)), ((), ())), preferred_element_type=jnp.float32)
    out_ref[...] = res + bo_ref[...]


def _project(ctx, wo, bo):
    b = ctx.shape[0]
    v = wo.shape[1]
    vblk = 4096
    out_t = pl.pallas_call(
        _proj_body,
        grid=(pl.cdiv(v, vblk),),
        in_specs=[
            pl.BlockSpec((b, D), lambda j: (0, 0)),
            pl.BlockSpec((D, vblk), lambda j: (0, j)),
            pl.BlockSpec((vblk, 1), lambda j: (j, 0)),
        ],
        out_specs=pl.BlockSpec((vblk, b), lambda j: (j, 0)),
        out_shape=jax.ShapeDtypeStruct((v, b), jnp.float32),
    )(ctx, wo.astype(jnp.bfloat16), bo.reshape(-1, 1))
    return out_t.T


def kernel(seq, embed, W1, b1, W2, b2, gamma, beta, Wg, bg, Wr, br, Wo, bo, memory):
    b, t = seq.shape
    idx = seq.astype(jnp.int32).reshape(-1)
    h_flat = _sc_gather(embed, idx)
    hid, si = _encoder(h_flat, W1, b1, W2, b2, gamma, beta, Wg, bg)
    hid3 = hid.reshape(b, t, D)
    ctx = _select_read(hid3, si.reshape(b, t), Wr, br, memory[0])
    return _project(ctx, Wo, bo)
